# baseline (speedup 1.0000x reference)
import functools

import jax
import jax.numpy as jnp
from jax import lax
from jax.experimental import pallas as pl
from jax.experimental.pallas import tpu as pltpu
from jax.experimental.pallas import tpu_sc as plsc

_N = 10000
_DIN = 128
_D = 128
_E = 320000

_NC = 2
_NT = 16
_NW = _NC * _NT
_C = 128
_S = 80
_EP = _NW * _S * _C
_NP = 10112
_RT = _NP // _NT

_BR = 2528

_NB = 2
_SUB = 40
_GH = _SUB // _NB
_TCH = _EP // _C

_S_FAST = 120
_S_SLOW = 40
_FAST_CID = 0


def _sc_mesh():
    return plsc.VectorSubcoreMesh(core_axis_name="c", subcore_axis_name="s")


def _sc_scatter(f, src2, dst2, zrows):
    def body(f_hbm, src_hbm, dst_hbm, z_hbm, out_hbm, srcv, dstv,
             r0, r1, acc, g0, g1, s0, s1):
        rows = (r0, r1)
        gsem = (g0, g1)
        ssem = (s0, s1)
        cid = lax.axis_index("c")
        sid = lax.axis_index("s")
        fast = cid == _FAST_CID
        n_me = jnp.where(fast, _S_FAST, _S_SLOW)
        start = jnp.where(fast, sid * _S_FAST,
                          _NT * _S_FAST + sid * _S_SLOW)
        pltpu.sync_copy(z_hbm, acc.at[pl.ds(sid * _RT, _RT)])
        plsc.subcore_barrier()

        def subslab(h, carry):
            off = pl.multiple_of(start + h * _SUB, 8)
            pltpu.sync_copy(src_hbm.at[pl.ds(off, _SUB)], srcv)
            pltpu.sync_copy(dst_hbm.at[pl.ds(off, _SUB)], dstv)
            for b in range(_NB):
                pltpu.async_copy(f_hbm.at[srcv.at[b]], rows[b], gsem[b])

            def outer(g, c):
                base = g * _NB
                for b in range(_NB):
                    pltpu.make_async_copy(
                        f_hbm.at[srcv.at[base + b]], rows[b], gsem[b]).wait()
                    pltpu.async_copy(
                        rows[b], acc.at[dstv.at[base + b]], ssem[b], add=True)
                for b in range(_NB):
                    pltpu.make_async_copy(
                        rows[b], acc.at[dstv.at[base + b]], ssem[b]).wait()

                    @pl.when(g < _GH - 1)
                    def _():
                        pltpu.async_copy(
                            f_hbm.at[srcv.at[base + _NB + b]], rows[b], gsem[b])

                return c

            lax.fori_loop(0, _GH, outer, 0)
            return carry

        lax.fori_loop(0, n_me // _SUB, subslab, 0)
        plsc.subcore_barrier()
        pltpu.sync_copy(acc.at[pl.ds(sid * _RT, _RT)],
                        out_hbm.at[cid, pl.ds(sid * _RT, _RT)])

    return pl.kernel(
        body,
        out_type=jax.ShapeDtypeStruct((_NC, _NP, _D), jnp.float32),
        mesh=_sc_mesh(),
        scratch_types=[
            pltpu.VMEM((_SUB, _C), jnp.int32),
            pltpu.VMEM((_SUB, _C), jnp.int32),
        ] + [pltpu.VMEM((_C, _D), jnp.float32)] * _NB + [
            pltpu.VMEM_SHARED((_NP, _D), jnp.float32),
        ] + [pltpu.SemaphoreType.DMA] * (2 * _NB),
    )(f, src2, dst2, zrows)


def _row_spec(width=_D):
    return pl.BlockSpec((_BR, width), lambda i: (i, 0))


def _full_spec(shape):
    nd = len(shape)
    return pl.BlockSpec(shape, lambda i: (0,) * nd)


def _k2(x_pad, degpart, w, b):
    def body(x_ref, dp_ref, w_ref, b_ref, u_ref, dinv_ref):
        i = pl.program_id(0)
        deg = dp_ref[0, :, 0:1] + dp_ref[1, :, 0:1] + 1.0
        rows = lax.broadcasted_iota(jnp.int32, (_BR, 1), 0) + i * _BR
        dinv = jnp.where(rows < _N, lax.rsqrt(deg), 0.0)
        phi = jnp.maximum(
            jnp.dot(x_ref[...], w_ref[...], preferred_element_type=jnp.float32)
            + b_ref[...][None, :], 0.0)
        u_ref[...] = dinv * phi
        dinv_ref[...] = dinv

    return pl.pallas_call(
        body,
        grid=(_NP // _BR,),
        in_specs=[
            _row_spec(_DIN),
            pl.BlockSpec((_NC, _BR, _D), lambda i: (0, i, 0)),
            _full_spec((_DIN, _D)),
            _full_spec((_D,)),
        ],
        out_specs=[_row_spec(_D), _row_spec(1)],
        out_shape=[
            jax.ShapeDtypeStruct((_NP, _D), jnp.float32),
            jax.ShapeDtypeStruct((_NP, 1), jnp.float32),
        ],
    )(x_pad, degpart, w, b)


def _k4(acc, u1, dinv, w, b):
    def body(a_ref, u_ref, d_ref, w_ref, b_ref, q_ref, e_ref):
        d = d_ref[...]
        q = d * (a_ref[0] + a_ref[1] + u_ref[...])
        enc = jnp.dot(q, w_ref[...], preferred_element_type=jnp.float32) + b_ref[...][None, :]
        q_ref[...] = q
        e_ref[...] = d * enc

    return pl.pallas_call(
        body,
        grid=(_NP // _BR,),
        in_specs=[
            pl.BlockSpec((_NC, _BR, _D), lambda i: (0, i, 0)),
            _row_spec(_D),
            _row_spec(1),
            _full_spec((_D, _D)),
            _full_spec((_D,)),
        ],
        out_specs=[_row_spec(_D), _row_spec(_D)],
        out_shape=[
            jax.ShapeDtypeStruct((_NP, _D), jnp.float32),
            jax.ShapeDtypeStruct((_NP, _D), jnp.float32),
        ],
    )(acc, u1, dinv, w, b)


def _k6(acc, e1, dinv, eps, wm, bm, ws, bs, wz, bz):
    def body(a_ref, e_ref, d_ref, eps_ref, wm_ref, bm_ref, ws_ref, bs_ref,
             wz_ref, bz_ref, mean_ref, std_ref, z_ref, p_ref):
        d = d_ref[...]
        r = d * (a_ref[0] + a_ref[1] + e_ref[...])
        m = jnp.dot(r, wm_ref[...], preferred_element_type=jnp.float32) + bm_ref[...][None, :]
        s = jax.nn.softplus(
            jnp.dot(r, ws_ref[...], preferred_element_type=jnp.float32) + bs_ref[...][None, :])
        z = eps_ref[...] * s + m
        pz = jnp.maximum(
            jnp.dot(z, wz_ref[...], preferred_element_type=jnp.float32) + bz_ref[...][None, :],
            0.0)
        mean_ref[...] = m
        std_ref[...] = s
        z_ref[...] = z
        p_ref[...] = d * pz

    return pl.pallas_call(
        body,
        grid=(_NP // _BR,),
        in_specs=[
            pl.BlockSpec((_NC, _BR, _D), lambda i: (0, i, 0)),
            _row_spec(_D),
            _row_spec(1),
            _row_spec(_D),
            _full_spec((_D, _D)),
            _full_spec((_D,)),
            _full_spec((_D, _D)),
            _full_spec((_D,)),
            _full_spec((_D, _D)),
            _full_spec((_D,)),
        ],
        out_specs=[_row_spec(_D)] * 4,
        out_shape=[jax.ShapeDtypeStruct((_NP, _D), jnp.float32)] * 4,
    )(acc, e1, dinv, eps, wm, bm, ws, bs, wz, bz)


def _k8(acc, p1, dinv, q, wxz_t, wxz_b, wxh_t, wxh_b, bxz, bhz, bxh, bhh,
        prior_b, wpm, bpm, wps, bps):
    def body(a_ref, p_ref, d_ref, q_ref, wzt_ref, wzb_ref, wht_ref, whb_ref,
             bxz_ref, bhz_ref, bxh_ref, bhh_ref, pb_ref, wpm_ref, bpm_ref,
             wps_ref, bps_ref, h_ref, pm_ref, ps_ref):
        d = d_ref[...]
        t = d * (a_ref[0] + a_ref[1] + p_ref[...])
        qv = q_ref[...]
        ga = (jnp.dot(qv, wzt_ref[...], preferred_element_type=jnp.float32)
              + jnp.dot(t, wzb_ref[...], preferred_element_type=jnp.float32)
              + (bxz_ref[...] + bhz_ref[...])[None, :])
        z_g = jax.nn.sigmoid(ga)
        ha = (jnp.dot(qv, wht_ref[...], preferred_element_type=jnp.float32)
              + jnp.dot(t, whb_ref[...], preferred_element_type=jnp.float32)
              + (bxh_ref[...] + bhh_ref[...])[None, :])
        h_tilde = jnp.tanh(ha)
        h_ref[...] = (1.0 - z_g) * h_tilde
        pr = jnp.maximum(pb_ref[...], 0.0)[None, :]
        pm_row = jnp.dot(pr, wpm_ref[...], preferred_element_type=jnp.float32) + bpm_ref[...][None, :]
        ps_row = jax.nn.softplus(
            jnp.dot(pr, wps_ref[...], preferred_element_type=jnp.float32) + bps_ref[...][None, :])
        pm_ref[...] = jnp.broadcast_to(pm_row, (_BR, _D))
        ps_ref[...] = jnp.broadcast_to(ps_row, (_BR, _D))

    return pl.pallas_call(
        body,
        grid=(_NP // _BR,),
        in_specs=[
            pl.BlockSpec((_NC, _BR, _D), lambda i: (0, i, 0)),
            _row_spec(_D),
            _row_spec(1),
            _row_spec(_D),
            _full_spec((_D, _D)),
            _full_spec((_D, _D)),
            _full_spec((_D, _D)),
            _full_spec((_D, _D)),
            _full_spec((_D,)),
            _full_spec((_D,)),
            _full_spec((_D,)),
            _full_spec((_D,)),
            _full_spec((_D,)),
            _full_spec((_D, _D)),
            _full_spec((_D,)),
            _full_spec((_D, _D)),
            _full_spec((_D,)),
        ],
        out_specs=[_row_spec(_D)] * 3,
        out_shape=[jax.ShapeDtypeStruct((_NP, _D), jnp.float32)] * 3,
    )(acc, p1, dinv, q, wxz_t, wxz_b, wxh_t, wxh_b, bxz, bhz, bxh, bhh,
      prior_b, wpm, bpm, wps, bps)


_BRD = 400


def _k9(zs):
    def body(a_ref, b_ref, o_ref):
        o_ref[...] = jax.nn.sigmoid(
            lax.dot_general(a_ref[...], b_ref[...], (((1,), (1,)), ((), ())),
                            preferred_element_type=jnp.float32))

    return pl.pallas_call(
        body,
        grid=(_N // _BRD,),
        in_specs=[
            pl.BlockSpec((_BRD, _D), lambda i: (i, 0)),
            pl.BlockSpec((_N, _D), lambda i: (0, 0)),
        ],
        out_specs=pl.BlockSpec((_BRD, _N), lambda i: (i, 0)),
        out_shape=jax.ShapeDtypeStruct((_N, _N), jnp.float32),
    )(zs, zs)


def kernel(x, edge_index, phi_x_W, phi_x_b, phi_z_W, phi_z_b, enc_W, enc_b,
           enc_mean_W, enc_mean_b, enc_std_W, enc_std_b, prior_W, prior_b,
           prior_mean_W, prior_mean_b, prior_std_W, prior_std_b, rnn_xz_W,
           rnn_xz_b, rnn_hz_W, rnn_hz_b, rnn_xr_W, rnn_xr_b, rnn_hr_W,
           rnn_hr_b, rnn_xh_W, rnn_xh_b, rnn_hh_W, rnn_hh_b):
    f32 = jnp.float32
    src = edge_index[0].astype(jnp.int32)
    dst = edge_index[1].astype(jnp.int32)
    padv = jnp.full((_EP - _E,), _N, jnp.int32)
    src2 = jnp.concatenate([src, padv]).reshape(_TCH, _C)
    dst2 = jnp.concatenate([dst, padv]).reshape(_TCH, _C)
    zrows = jnp.zeros((_RT, _D), f32)
    ones_mat = jnp.ones((_NP, _D), f32)
    zpad = jnp.zeros((_NP - _N, _D), f32)

    src_deg = jnp.full((_TCH, _C), _N, jnp.int32)
    degpart = _sc_scatter(ones_mat, src_deg, dst2, zrows)

    x_pad = jnp.concatenate([x.astype(f32), zpad], axis=0)
    u1, dinv = _k2(x_pad, degpart, phi_x_W, phi_x_b)

    acc_a = _sc_scatter(u1, src2, dst2, zrows)
    q, e1 = _k4(acc_a, u1, dinv, enc_W[:_D], enc_b)

    acc_b = _sc_scatter(e1, src2, dst2, zrows)
    eps = jax.random.normal(jax.random.key(42), (_N, _D), dtype=f32)
    eps_pad = jnp.concatenate([eps, zpad], axis=0)
    enc_mean_p, enc_std_p, z_p, p1 = _k6(
        acc_b, e1, dinv, eps_pad, enc_mean_W, enc_mean_b, enc_std_W,
        enc_std_b, phi_z_W, phi_z_b)

    acc_c = _sc_scatter(p1, src2, dst2, zrows)
    h_p, pm_p, ps_p = _k8(
        acc_c, p1, dinv, q, rnn_xz_W[:_D], rnn_xz_W[_D:], rnn_xh_W[:_D],
        rnn_xh_W[_D:], rnn_xz_b, rnn_hz_b, rnn_xh_b, rnn_hh_b, prior_b,
        prior_mean_W, prior_mean_b, prior_std_W, prior_std_b)

    zs = z_p[:_N]
    dec = _k9(zs)

    return (dec, enc_mean_p[:_N], enc_std_p[:_N], pm_p[:_N], ps_p[:_N],
            h_p[:_N][None])


# speedup vs baseline: 6.0170x; 6.0170x over previous
import functools

import jax
import jax.numpy as jnp
from jax import lax
from jax.experimental import pallas as pl
from jax.experimental.pallas import tpu as pltpu
from jax.experimental.pallas import tpu_sc as plsc

_N = 10000
_DIN = 128
_D = 128
_E = 320000

_NC = 2
_NT = 16
_NW = _NC * _NT
_C = 128
_S = 80
_EP = _NW * _S * _C
_NP = 10112
_RT = _NP // _NT

_BR = 2528

_NB = 2
_SUB = 40
_GH = _SUB // _NB
_TCH = _EP // _C

_S_FAST = 160
_S_SLOW = 0
_FAST_CID = 0


def _sc_mesh():
    return plsc.VectorSubcoreMesh(core_axis_name="c", subcore_axis_name="s")


def _sc_scatter(f, src2, dst2, zrows):
    def body(f_hbm, src_hbm, dst_hbm, z_hbm, out_hbm, srcv, dstv,
             r0, r1, acc, g0, g1, s0, s1):
        rows = (r0, r1)
        gsem = (g0, g1)
        ssem = (s0, s1)
        cid = lax.axis_index("c")
        sid = lax.axis_index("s")
        fast = cid == _FAST_CID
        n_me = jnp.where(fast, _S_FAST, _S_SLOW)
        start = jnp.where(fast, sid * _S_FAST,
                          _NT * _S_FAST + sid * _S_SLOW)
        pltpu.sync_copy(z_hbm, acc.at[pl.ds(sid * _RT, _RT)])
        plsc.subcore_barrier()

        def subslab(h, carry):
            off = pl.multiple_of(start + h * _SUB, 8)
            pltpu.sync_copy(src_hbm.at[pl.ds(off, _SUB)], srcv)
            pltpu.sync_copy(dst_hbm.at[pl.ds(off, _SUB)], dstv)
            for b in range(_NB):
                pltpu.async_copy(f_hbm.at[srcv.at[b]], rows[b], gsem[b])

            def outer(g, c):
                base = g * _NB
                for b in range(_NB):
                    pltpu.make_async_copy(
                        f_hbm.at[srcv.at[base + b]], rows[b], gsem[b]).wait()
                    pltpu.async_copy(
                        rows[b], acc.at[dstv.at[base + b]], ssem[b], add=True)
                for b in range(_NB):
                    pltpu.make_async_copy(
                        rows[b], acc.at[dstv.at[base + b]], ssem[b]).wait()

                    @pl.when(g < _GH - 1)
                    def _():
                        pltpu.async_copy(
                            f_hbm.at[srcv.at[base + _NB + b]], rows[b], gsem[b])

                return c

            lax.fori_loop(0, _GH, outer, 0)
            return carry

        lax.fori_loop(0, n_me // _SUB, subslab, 0)
        plsc.subcore_barrier()
        pltpu.sync_copy(acc.at[pl.ds(sid * _RT, _RT)],
                        out_hbm.at[cid, pl.ds(sid * _RT, _RT)])

    return pl.kernel(
        body,
        out_type=jax.ShapeDtypeStruct((_NC, _NP, _D), jnp.float32),
        mesh=_sc_mesh(),
        scratch_types=[
            pltpu.VMEM((_SUB, _C), jnp.int32),
            pltpu.VMEM((_SUB, _C), jnp.int32),
        ] + [pltpu.VMEM((_C, _D), jnp.float32)] * _NB + [
            pltpu.VMEM_SHARED((_NP, _D), jnp.float32),
        ] + [pltpu.SemaphoreType.DMA] * (2 * _NB),
    )(f, src2, dst2, zrows)


def _row_spec(width=_D):
    return pl.BlockSpec((_BR, width), lambda i: (i, 0))


def _full_spec(shape):
    nd = len(shape)
    return pl.BlockSpec(shape, lambda i: (0,) * nd)


def _k2(x_pad, degpart, w, b):
    def body(x_ref, dp_ref, w_ref, b_ref, u_ref, dinv_ref):
        i = pl.program_id(0)
        deg = dp_ref[0, :, 0:1] + dp_ref[1, :, 0:1] + 1.0
        rows = lax.broadcasted_iota(jnp.int32, (_BR, 1), 0) + i * _BR
        dinv = jnp.where(rows < _N, lax.rsqrt(deg), 0.0)
        phi = jnp.maximum(
            jnp.dot(x_ref[...], w_ref[...], preferred_element_type=jnp.float32)
            + b_ref[...][None, :], 0.0)
        u_ref[...] = dinv * phi
        dinv_ref[...] = dinv

    return pl.pallas_call(
        body,
        grid=(_NP // _BR,),
        in_specs=[
            _row_spec(_DIN),
            pl.BlockSpec((_NC, _BR, _D), lambda i: (0, i, 0)),
            _full_spec((_DIN, _D)),
            _full_spec((_D,)),
        ],
        out_specs=[_row_spec(_D), _row_spec(1)],
        out_shape=[
            jax.ShapeDtypeStruct((_NP, _D), jnp.float32),
            jax.ShapeDtypeStruct((_NP, 1), jnp.float32),
        ],
    )(x_pad, degpart, w, b)


def _k4(acc, u1, dinv, w, b):
    def body(a_ref, u_ref, d_ref, w_ref, b_ref, q_ref, e_ref):
        d = d_ref[...]
        q = d * (a_ref[0] + a_ref[1] + u_ref[...])
        enc = jnp.dot(q, w_ref[...], preferred_element_type=jnp.float32) + b_ref[...][None, :]
        q_ref[...] = q
        e_ref[...] = d * enc

    return pl.pallas_call(
        body,
        grid=(_NP // _BR,),
        in_specs=[
            pl.BlockSpec((_NC, _BR, _D), lambda i: (0, i, 0)),
            _row_spec(_D),
            _row_spec(1),
            _full_spec((_D, _D)),
            _full_spec((_D,)),
        ],
        out_specs=[_row_spec(_D), _row_spec(_D)],
        out_shape=[
            jax.ShapeDtypeStruct((_NP, _D), jnp.float32),
            jax.ShapeDtypeStruct((_NP, _D), jnp.float32),
        ],
    )(acc, u1, dinv, w, b)


def _k6(acc, e1, dinv, eps, wm, bm, ws, bs, wz, bz):
    def body(a_ref, e_ref, d_ref, eps_ref, wm_ref, bm_ref, ws_ref, bs_ref,
             wz_ref, bz_ref, mean_ref, std_ref, z_ref, p_ref):
        d = d_ref[...]
        r = d * (a_ref[0] + a_ref[1] + e_ref[...])
        m = jnp.dot(r, wm_ref[...], preferred_element_type=jnp.float32) + bm_ref[...][None, :]
        s = jax.nn.softplus(
            jnp.dot(r, ws_ref[...], preferred_element_type=jnp.float32) + bs_ref[...][None, :])
        z = eps_ref[...] * s + m
        pz = jnp.maximum(
            jnp.dot(z, wz_ref[...], preferred_element_type=jnp.float32) + bz_ref[...][None, :],
            0.0)
        mean_ref[...] = m
        std_ref[...] = s
        z_ref[...] = z
        p_ref[...] = d * pz

    return pl.pallas_call(
        body,
        grid=(_NP // _BR,),
        in_specs=[
            pl.BlockSpec((_NC, _BR, _D), lambda i: (0, i, 0)),
            _row_spec(_D),
            _row_spec(1),
            _row_spec(_D),
            _full_spec((_D, _D)),
            _full_spec((_D,)),
            _full_spec((_D, _D)),
            _full_spec((_D,)),
            _full_spec((_D, _D)),
            _full_spec((_D,)),
        ],
        out_specs=[_row_spec(_D)] * 4,
        out_shape=[jax.ShapeDtypeStruct((_NP, _D), jnp.float32)] * 4,
    )(acc, e1, dinv, eps, wm, bm, ws, bs, wz, bz)


def _k8(acc, p1, dinv, q, wxz_t, wxz_b, wxh_t, wxh_b, bxz, bhz, bxh, bhh,
        prior_b, wpm, bpm, wps, bps):
    def body(a_ref, p_ref, d_ref, q_ref, wzt_ref, wzb_ref, wht_ref, whb_ref,
             bxz_ref, bhz_ref, bxh_ref, bhh_ref, pb_ref, wpm_ref, bpm_ref,
             wps_ref, bps_ref, h_ref, pm_ref, ps_ref):
        d = d_ref[...]
        t = d * (a_ref[0] + a_ref[1] + p_ref[...])
        qv = q_ref[...]
        ga = (jnp.dot(qv, wzt_ref[...], preferred_element_type=jnp.float32)
              + jnp.dot(t, wzb_ref[...], preferred_element_type=jnp.float32)
              + (bxz_ref[...] + bhz_ref[...])[None, :])
        z_g = jax.nn.sigmoid(ga)
        ha = (jnp.dot(qv, wht_ref[...], preferred_element_type=jnp.float32)
              + jnp.dot(t, whb_ref[...], preferred_element_type=jnp.float32)
              + (bxh_ref[...] + bhh_ref[...])[None, :])
        h_tilde = jnp.tanh(ha)
        h_ref[...] = (1.0 - z_g) * h_tilde
        pr = jnp.maximum(pb_ref[...], 0.0)[None, :]
        pm_row = jnp.dot(pr, wpm_ref[...], preferred_element_type=jnp.float32) + bpm_ref[...][None, :]
        ps_row = jax.nn.softplus(
            jnp.dot(pr, wps_ref[...], preferred_element_type=jnp.float32) + bps_ref[...][None, :])
        pm_ref[...] = jnp.broadcast_to(pm_row, (_BR, _D))
        ps_ref[...] = jnp.broadcast_to(ps_row, (_BR, _D))

    return pl.pallas_call(
        body,
        grid=(_NP // _BR,),
        in_specs=[
            pl.BlockSpec((_NC, _BR, _D), lambda i: (0, i, 0)),
            _row_spec(_D),
            _row_spec(1),
            _row_spec(_D),
            _full_spec((_D, _D)),
            _full_spec((_D, _D)),
            _full_spec((_D, _D)),
            _full_spec((_D, _D)),
            _full_spec((_D,)),
            _full_spec((_D,)),
            _full_spec((_D,)),
            _full_spec((_D,)),
            _full_spec((_D,)),
            _full_spec((_D, _D)),
            _full_spec((_D,)),
            _full_spec((_D, _D)),
            _full_spec((_D,)),
        ],
        out_specs=[_row_spec(_D)] * 3,
        out_shape=[jax.ShapeDtypeStruct((_NP, _D), jnp.float32)] * 3,
    )(acc, p1, dinv, q, wxz_t, wxz_b, wxh_t, wxh_b, bxz, bhz, bxh, bhh,
      prior_b, wpm, bpm, wps, bps)


_BRD = 400


def _k9(zs):
    def body(a_ref, b_ref, o_ref):
        o_ref[...] = jax.nn.sigmoid(
            lax.dot_general(a_ref[...], b_ref[...], (((1,), (1,)), ((), ())),
                            preferred_element_type=jnp.float32))

    return pl.pallas_call(
        body,
        grid=(_N // _BRD,),
        in_specs=[
            pl.BlockSpec((_BRD, _D), lambda i: (i, 0)),
            pl.BlockSpec((_N, _D), lambda i: (0, 0)),
        ],
        out_specs=pl.BlockSpec((_BRD, _N), lambda i: (i, 0)),
        out_shape=jax.ShapeDtypeStruct((_N, _N), jnp.float32),
    )(zs, zs)


def kernel(x, edge_index, phi_x_W, phi_x_b, phi_z_W, phi_z_b, enc_W, enc_b,
           enc_mean_W, enc_mean_b, enc_std_W, enc_std_b, prior_W, prior_b,
           prior_mean_W, prior_mean_b, prior_std_W, prior_std_b, rnn_xz_W,
           rnn_xz_b, rnn_hz_W, rnn_hz_b, rnn_xr_W, rnn_xr_b, rnn_hr_W,
           rnn_hr_b, rnn_xh_W, rnn_xh_b, rnn_hh_W, rnn_hh_b):
    f32 = jnp.float32
    src = edge_index[0].astype(jnp.int32)
    dst = edge_index[1].astype(jnp.int32)
    padv = jnp.full((_EP - _E,), _N, jnp.int32)
    src2 = jnp.concatenate([src, padv]).reshape(_TCH, _C)
    dst2 = jnp.concatenate([dst, padv]).reshape(_TCH, _C)
    zrows = jnp.zeros((_RT, _D), f32)
    ones_mat = jnp.ones((_NP, _D), f32)
    zpad = jnp.zeros((_NP - _N, _D), f32)

    src_deg = (jnp.arange(_EP, dtype=jnp.int32) % _NP).reshape(_TCH, _C)
    degpart = _sc_scatter(ones_mat, src_deg, dst2, zrows)

    x_pad = jnp.concatenate([x.astype(f32), zpad], axis=0)
    u1, dinv = _k2(x_pad, degpart, phi_x_W, phi_x_b)

    acc_a = _sc_scatter(u1, src2, dst2, zrows)
    q, e1 = _k4(acc_a, u1, dinv, enc_W[:_D], enc_b)

    acc_b = _sc_scatter(e1, src2, dst2, zrows)
    eps = jax.random.normal(jax.random.key(42), (_N, _D), dtype=f32)
    eps_pad = jnp.concatenate([eps, zpad], axis=0)
    enc_mean_p, enc_std_p, z_p, p1 = _k6(
        acc_b, e1, dinv, eps_pad, enc_mean_W, enc_mean_b, enc_std_W,
        enc_std_b, phi_z_W, phi_z_b)

    acc_c = _sc_scatter(p1, src2, dst2, zrows)
    h_p, pm_p, ps_p = _k8(
        acc_c, p1, dinv, q, rnn_xz_W[:_D], rnn_xz_W[_D:], rnn_xh_W[:_D],
        rnn_xh_W[_D:], rnn_xz_b, rnn_hz_b, rnn_xh_b, rnn_hh_b, prior_b,
        prior_mean_W, prior_mean_b, prior_std_W, prior_std_b)

    zs = z_p[:_N]
    dec = _k9(zs)

    return (dec, enc_mean_p[:_N], enc_std_p[:_N], pm_p[:_N], ps_p[:_N],
            h_p[:_N][None])


# speedup vs baseline: 8.2014x; 1.3630x over previous
import functools

import jax
import jax.numpy as jnp
from jax import lax
from jax.experimental import pallas as pl
from jax.experimental.pallas import tpu as pltpu
from jax.experimental.pallas import tpu_sc as plsc

_N = 10000
_DIN = 128
_D = 128
_E = 320000

_NC = 2
_NT = 16
_NW = _NC * _NT
_C = 128
_S = 80
_EP = _NW * _S * _C
_NP = 10112
_RT = _NP // _NT

_BR = 2528

_NB = 2
_SUB = 40
_GH = _SUB // _NB
_TCH = _EP // _C

_S_FAST = 120
_S_SLOW = 40
_FAST_CID = 0


def _sc_mesh():
    return plsc.VectorSubcoreMesh(core_axis_name="c", subcore_axis_name="s")


def _sc_degree(dst2, ones_rows, zrows):
    def body(dst_hbm, ones_hbm, z_hbm, out_hbm, dstv, onesv, acc):
        cid = lax.axis_index("c")
        sid = lax.axis_index("s")
        wid = sid * _NC + cid
        pltpu.sync_copy(z_hbm, acc.at[pl.ds(sid * _RT, _RT)])
        pltpu.sync_copy(ones_hbm, onesv)
        pltpu.sync_copy(dst_hbm.at[pl.ds(wid * _S, _S)], dstv)
        plsc.subcore_barrier()

        def step(k, carry):
            pltpu.sync_copy(onesv, acc.at[dstv.at[k]], add=True)
            return carry

        lax.fori_loop(0, _S, step, 0)
        plsc.subcore_barrier()
        pltpu.sync_copy(acc.at[pl.ds(sid * _RT, _RT)],
                        out_hbm.at[cid, pl.ds(sid * _RT, _RT)])

    return pl.kernel(
        body,
        out_type=jax.ShapeDtypeStruct((_NC, _NP, _D), jnp.float32),
        mesh=_sc_mesh(),
        scratch_types=[
            pltpu.VMEM((_S, _C), jnp.int32),
            pltpu.VMEM((_C, _D), jnp.float32),
            pltpu.VMEM_SHARED((_NP, _D), jnp.float32),
        ],
    )(dst2, ones_rows, zrows)


def _sc_scatter(f, src2, dst2, zrows):
    def body(f_hbm, src_hbm, dst_hbm, z_hbm, out_hbm, srcv, dstv,
             r0, r1, acc, g0, g1, s0, s1):
        rows = (r0, r1)
        gsem = (g0, g1)
        ssem = (s0, s1)
        cid = lax.axis_index("c")
        sid = lax.axis_index("s")
        fast = cid == _FAST_CID
        n_me = jnp.where(fast, _S_FAST, _S_SLOW)
        start = jnp.where(fast, sid * _S_FAST,
                          _NT * _S_FAST + sid * _S_SLOW)
        pltpu.sync_copy(z_hbm, acc.at[pl.ds(sid * _RT, _RT)])
        plsc.subcore_barrier()

        def subslab(h, carry):
            off = pl.multiple_of(start + h * _SUB, 8)
            pltpu.sync_copy(src_hbm.at[pl.ds(off, _SUB)], srcv)
            pltpu.sync_copy(dst_hbm.at[pl.ds(off, _SUB)], dstv)
            for b in range(_NB):
                pltpu.async_copy(f_hbm.at[srcv.at[b]], rows[b], gsem[b])

            def outer(g, c):
                base = g * _NB
                for b in range(_NB):
                    pltpu.make_async_copy(
                        f_hbm.at[srcv.at[base + b]], rows[b], gsem[b]).wait()
                    pltpu.async_copy(
                        rows[b], acc.at[dstv.at[base + b]], ssem[b], add=True)
                for b in range(_NB):
                    pltpu.make_async_copy(
                        rows[b], acc.at[dstv.at[base + b]], ssem[b]).wait()

                    @pl.when(g < _GH - 1)
                    def _():
                        pltpu.async_copy(
                            f_hbm.at[srcv.at[base + _NB + b]], rows[b], gsem[b])

                return c

            lax.fori_loop(0, _GH, outer, 0)
            return carry

        lax.fori_loop(0, n_me // _SUB, subslab, 0)
        plsc.subcore_barrier()
        pltpu.sync_copy(acc.at[pl.ds(sid * _RT, _RT)],
                        out_hbm.at[cid, pl.ds(sid * _RT, _RT)])

    return pl.kernel(
        body,
        out_type=jax.ShapeDtypeStruct((_NC, _NP, _D), jnp.float32),
        mesh=_sc_mesh(),
        scratch_types=[
            pltpu.VMEM((_SUB, _C), jnp.int32),
            pltpu.VMEM((_SUB, _C), jnp.int32),
        ] + [pltpu.VMEM((_C, _D), jnp.float32)] * _NB + [
            pltpu.VMEM_SHARED((_NP, _D), jnp.float32),
        ] + [pltpu.SemaphoreType.DMA] * (2 * _NB),
    )(f, src2, dst2, zrows)


def _row_spec(width=_D):
    return pl.BlockSpec((_BR, width), lambda i: (i, 0))


def _full_spec(shape):
    nd = len(shape)
    return pl.BlockSpec(shape, lambda i: (0,) * nd)


def _k2(x_pad, degpart, w, b):
    def body(x_ref, dp_ref, w_ref, b_ref, u_ref, dinv_ref):
        i = pl.program_id(0)
        deg = dp_ref[0, :, 0:1] + dp_ref[1, :, 0:1] + 1.0
        rows = lax.broadcasted_iota(jnp.int32, (_BR, 1), 0) + i * _BR
        dinv = jnp.where(rows < _N, lax.rsqrt(deg), 0.0)
        phi = jnp.maximum(
            jnp.dot(x_ref[...], w_ref[...], preferred_element_type=jnp.float32)
            + b_ref[...][None, :], 0.0)
        u_ref[...] = dinv * phi
        dinv_ref[...] = dinv

    return pl.pallas_call(
        body,
        grid=(_NP // _BR,),
        in_specs=[
            _row_spec(_DIN),
            pl.BlockSpec((_NC, _BR, _D), lambda i: (0, i, 0)),
            _full_spec((_DIN, _D)),
            _full_spec((_D,)),
        ],
        out_specs=[_row_spec(_D), _row_spec(1)],
        out_shape=[
            jax.ShapeDtypeStruct((_NP, _D), jnp.float32),
            jax.ShapeDtypeStruct((_NP, 1), jnp.float32),
        ],
    )(x_pad, degpart, w, b)


def _k4(acc, u1, dinv, w, b):
    def body(a_ref, u_ref, d_ref, w_ref, b_ref, q_ref, e_ref):
        d = d_ref[...]
        q = d * (a_ref[0] + a_ref[1] + u_ref[...])
        enc = jnp.dot(q, w_ref[...], preferred_element_type=jnp.float32) + b_ref[...][None, :]
        q_ref[...] = q
        e_ref[...] = d * enc

    return pl.pallas_call(
        body,
        grid=(_NP // _BR,),
        in_specs=[
            pl.BlockSpec((_NC, _BR, _D), lambda i: (0, i, 0)),
            _row_spec(_D),
            _row_spec(1),
            _full_spec((_D, _D)),
            _full_spec((_D,)),
        ],
        out_specs=[_row_spec(_D), _row_spec(_D)],
        out_shape=[
            jax.ShapeDtypeStruct((_NP, _D), jnp.float32),
            jax.ShapeDtypeStruct((_NP, _D), jnp.float32),
        ],
    )(acc, u1, dinv, w, b)


def _k6(acc, e1, dinv, eps, wm, bm, ws, bs, wz, bz):
    def body(a_ref, e_ref, d_ref, eps_ref, wm_ref, bm_ref, ws_ref, bs_ref,
             wz_ref, bz_ref, mean_ref, std_ref, z_ref, p_ref):
        d = d_ref[...]
        r = d * (a_ref[0] + a_ref[1] + e_ref[...])
        m = jnp.dot(r, wm_ref[...], preferred_element_type=jnp.float32) + bm_ref[...][None, :]
        s = jax.nn.softplus(
            jnp.dot(r, ws_ref[...], preferred_element_type=jnp.float32) + bs_ref[...][None, :])
        z = eps_ref[...] * s + m
        pz = jnp.maximum(
            jnp.dot(z, wz_ref[...], preferred_element_type=jnp.float32) + bz_ref[...][None, :],
            0.0)
        mean_ref[...] = m
        std_ref[...] = s
        z_ref[...] = z
        p_ref[...] = d * pz

    return pl.pallas_call(
        body,
        grid=(_NP // _BR,),
        in_specs=[
            pl.BlockSpec((_NC, _BR, _D), lambda i: (0, i, 0)),
            _row_spec(_D),
            _row_spec(1),
            _row_spec(_D),
            _full_spec((_D, _D)),
            _full_spec((_D,)),
            _full_spec((_D, _D)),
            _full_spec((_D,)),
            _full_spec((_D, _D)),
            _full_spec((_D,)),
        ],
        out_specs=[_row_spec(_D)] * 4,
        out_shape=[jax.ShapeDtypeStruct((_NP, _D), jnp.float32)] * 4,
    )(acc, e1, dinv, eps, wm, bm, ws, bs, wz, bz)


def _k8(acc, p1, dinv, q, wxz_t, wxz_b, wxh_t, wxh_b, bxz, bhz, bxh, bhh,
        prior_b, wpm, bpm, wps, bps):
    def body(a_ref, p_ref, d_ref, q_ref, wzt_ref, wzb_ref, wht_ref, whb_ref,
             bxz_ref, bhz_ref, bxh_ref, bhh_ref, pb_ref, wpm_ref, bpm_ref,
             wps_ref, bps_ref, h_ref, pm_ref, ps_ref):
        d = d_ref[...]
        t = d * (a_ref[0] + a_ref[1] + p_ref[...])
        qv = q_ref[...]
        ga = (jnp.dot(qv, wzt_ref[...], preferred_element_type=jnp.float32)
              + jnp.dot(t, wzb_ref[...], preferred_element_type=jnp.float32)
              + (bxz_ref[...] + bhz_ref[...])[None, :])
        z_g = jax.nn.sigmoid(ga)
        ha = (jnp.dot(qv, wht_ref[...], preferred_element_type=jnp.float32)
              + jnp.dot(t, whb_ref[...], preferred_element_type=jnp.float32)
              + (bxh_ref[...] + bhh_ref[...])[None, :])
        h_tilde = jnp.tanh(ha)
        h_ref[...] = (1.0 - z_g) * h_tilde
        pr = jnp.maximum(pb_ref[...], 0.0)[None, :]
        pm_row = jnp.dot(pr, wpm_ref[...], preferred_element_type=jnp.float32) + bpm_ref[...][None, :]
        ps_row = jax.nn.softplus(
            jnp.dot(pr, wps_ref[...], preferred_element_type=jnp.float32) + bps_ref[...][None, :])
        pm_ref[...] = jnp.broadcast_to(pm_row, (_BR, _D))
        ps_ref[...] = jnp.broadcast_to(ps_row, (_BR, _D))

    return pl.pallas_call(
        body,
        grid=(_NP // _BR,),
        in_specs=[
            pl.BlockSpec((_NC, _BR, _D), lambda i: (0, i, 0)),
            _row_spec(_D),
            _row_spec(1),
            _row_spec(_D),
            _full_spec((_D, _D)),
            _full_spec((_D, _D)),
            _full_spec((_D, _D)),
            _full_spec((_D, _D)),
            _full_spec((_D,)),
            _full_spec((_D,)),
            _full_spec((_D,)),
            _full_spec((_D,)),
            _full_spec((_D,)),
            _full_spec((_D, _D)),
            _full_spec((_D,)),
            _full_spec((_D, _D)),
            _full_spec((_D,)),
        ],
        out_specs=[_row_spec(_D)] * 3,
        out_shape=[jax.ShapeDtypeStruct((_NP, _D), jnp.float32)] * 3,
    )(acc, p1, dinv, q, wxz_t, wxz_b, wxh_t, wxh_b, bxz, bhz, bxh, bhh,
      prior_b, wpm, bpm, wps, bps)


_BRD = 400


def _k9(zs):
    def body(a_ref, b_ref, o_ref):
        o_ref[...] = jax.nn.sigmoid(
            lax.dot_general(a_ref[...], b_ref[...], (((1,), (1,)), ((), ())),
                            preferred_element_type=jnp.float32))

    return pl.pallas_call(
        body,
        grid=(_N // _BRD,),
        in_specs=[
            pl.BlockSpec((_BRD, _D), lambda i: (i, 0)),
            pl.BlockSpec((_N, _D), lambda i: (0, 0)),
        ],
        out_specs=pl.BlockSpec((_BRD, _N), lambda i: (i, 0)),
        out_shape=jax.ShapeDtypeStruct((_N, _N), jnp.float32),
    )(zs, zs)


def kernel(x, edge_index, phi_x_W, phi_x_b, phi_z_W, phi_z_b, enc_W, enc_b,
           enc_mean_W, enc_mean_b, enc_std_W, enc_std_b, prior_W, prior_b,
           prior_mean_W, prior_mean_b, prior_std_W, prior_std_b, rnn_xz_W,
           rnn_xz_b, rnn_hz_W, rnn_hz_b, rnn_xr_W, rnn_xr_b, rnn_hr_W,
           rnn_hr_b, rnn_xh_W, rnn_xh_b, rnn_hh_W, rnn_hh_b):
    f32 = jnp.float32
    src = edge_index[0].astype(jnp.int32)
    dst = edge_index[1].astype(jnp.int32)
    padv = jnp.full((_EP - _E,), _N, jnp.int32)
    src2 = jnp.concatenate([src, padv]).reshape(_TCH, _C)
    dst2 = jnp.concatenate([dst, padv]).reshape(_TCH, _C)
    zrows = jnp.zeros((_RT, _D), f32)
    zpad = jnp.zeros((_NP - _N, _D), f32)

    ones_rows = jnp.ones((_C, _D), f32)
    degpart = _sc_degree(dst2, ones_rows, zrows)

    x_pad = jnp.concatenate([x.astype(f32), zpad], axis=0)
    u1, dinv = _k2(x_pad, degpart, phi_x_W, phi_x_b)

    acc_a = _sc_scatter(u1, src2, dst2, zrows)
    q, e1 = _k4(acc_a, u1, dinv, enc_W[:_D], enc_b)

    acc_b = _sc_scatter(e1, src2, dst2, zrows)
    eps = jax.random.normal(jax.random.key(42), (_N, _D), dtype=f32)
    eps_pad = jnp.concatenate([eps, zpad], axis=0)
    enc_mean_p, enc_std_p, z_p, p1 = _k6(
        acc_b, e1, dinv, eps_pad, enc_mean_W, enc_mean_b, enc_std_W,
        enc_std_b, phi_z_W, phi_z_b)

    acc_c = _sc_scatter(p1, src2, dst2, zrows)
    h_p, pm_p, ps_p = _k8(
        acc_c, p1, dinv, q, rnn_xz_W[:_D], rnn_xz_W[_D:], rnn_xh_W[:_D],
        rnn_xh_W[_D:], rnn_xz_b, rnn_hz_b, rnn_xh_b, rnn_hh_b, prior_b,
        prior_mean_W, prior_mean_b, prior_std_W, prior_std_b)

    zs = z_p[:_N]
    dec = _k9(zs)

    return (dec, enc_mean_p[:_N], enc_std_p[:_N], pm_p[:_N], ps_p[:_N],
            h_p[:_N][None])


# speedup vs baseline: 9.6635x; 1.1783x over previous
import functools

import jax
import jax.numpy as jnp
from jax import lax
from jax.experimental import pallas as pl
from jax.experimental.pallas import tpu as pltpu
from jax.experimental.pallas import tpu_sc as plsc

_N = 10000
_DIN = 128
_D = 128
_E = 320000

_NC = 2
_NT = 16
_NW = _NC * _NT
_C = 128
_S = 80
_EP = _NW * _S * _C
_NP = 10112
_RT = _NP // _NT

_BR = 2528

_NB = 2
_TCH = _EP // _C

_S_FAST = 144
_S_SLOW = 16
_SUB_FAST = 48
_SUB_SLOW = 16
_FAST_CID = 0


def _sc_mesh():
    return plsc.VectorSubcoreMesh(core_axis_name="c", subcore_axis_name="s")


def _sc_degree(dst2, ones_rows, zrows):
    def body(dst_hbm, ones_hbm, z_hbm, out_hbm, dstv, onesv, acc):
        cid = lax.axis_index("c")
        sid = lax.axis_index("s")
        wid = sid * _NC + cid
        pltpu.sync_copy(z_hbm, acc.at[pl.ds(sid * _RT, _RT)])
        pltpu.sync_copy(ones_hbm, onesv)
        pltpu.sync_copy(dst_hbm.at[pl.ds(wid * _S, _S)], dstv)
        plsc.subcore_barrier()

        def step(k, carry):
            pltpu.sync_copy(onesv, acc.at[dstv.at[k]], add=True)
            return carry

        lax.fori_loop(0, _S, step, 0)
        plsc.subcore_barrier()
        pltpu.sync_copy(acc.at[pl.ds(sid * _RT, _RT)],
                        out_hbm.at[cid, pl.ds(sid * _RT, _RT)])

    return pl.kernel(
        body,
        out_type=jax.ShapeDtypeStruct((_NC, _NP, _D), jnp.float32),
        mesh=_sc_mesh(),
        scratch_types=[
            pltpu.VMEM((_S, _C), jnp.int32),
            pltpu.VMEM((_C, _D), jnp.float32),
            pltpu.VMEM_SHARED((_NP, _D), jnp.float32),
        ],
    )(dst2, ones_rows, zrows)


def _sc_scatter(f, src2, dst2, zrows):
    def body(f_hbm, src_hbm, dst_hbm, z_hbm, out_hbm, srcv, dstv,
             r0, r1, acc, g0, g1, s0, s1):
        rows = (r0, r1)
        gsem = (g0, g1)
        ssem = (s0, s1)
        cid = lax.axis_index("c")
        sid = lax.axis_index("s")
        fast = cid == _FAST_CID
        pltpu.sync_copy(z_hbm, acc.at[pl.ds(sid * _RT, _RT)])
        plsc.subcore_barrier()

        def run(n_chunks, sub, start):
            gh = sub // _NB

            def subslab(h, carry):
                off = pl.multiple_of(start + h * sub, 8)
                pltpu.sync_copy(src_hbm.at[pl.ds(off, sub)],
                                srcv.at[pl.ds(0, sub)])
                pltpu.sync_copy(dst_hbm.at[pl.ds(off, sub)],
                                dstv.at[pl.ds(0, sub)])
                for b in range(_NB):
                    pltpu.async_copy(f_hbm.at[srcv.at[b]], rows[b], gsem[b])

                def outer(g, c):
                    base = g * _NB
                    for b in range(_NB):
                        pltpu.make_async_copy(
                            f_hbm.at[srcv.at[base + b]], rows[b], gsem[b]).wait()
                        pltpu.async_copy(
                            rows[b], acc.at[dstv.at[base + b]], ssem[b], add=True)
                    for b in range(_NB):
                        pltpu.make_async_copy(
                            rows[b], acc.at[dstv.at[base + b]], ssem[b]).wait()

                        @pl.when(g < gh - 1)
                        def _():
                            pltpu.async_copy(
                                f_hbm.at[srcv.at[base + _NB + b]], rows[b],
                                gsem[b])

                    return c

                lax.fori_loop(0, gh, outer, 0)
                return carry

            lax.fori_loop(0, n_chunks // sub, subslab, 0)

        @pl.when(fast)
        def _():
            run(_S_FAST, _SUB_FAST, sid * _S_FAST)

        @pl.when(jnp.logical_not(fast))
        def _():
            run(_S_SLOW, _SUB_SLOW, _NT * _S_FAST + sid * _S_SLOW)

        plsc.subcore_barrier()
        pltpu.sync_copy(acc.at[pl.ds(sid * _RT, _RT)],
                        out_hbm.at[cid, pl.ds(sid * _RT, _RT)])

    return pl.kernel(
        body,
        out_type=jax.ShapeDtypeStruct((_NC, _NP, _D), jnp.float32),
        mesh=_sc_mesh(),
        scratch_types=[
            pltpu.VMEM((_SUB_FAST, _C), jnp.int32),
            pltpu.VMEM((_SUB_FAST, _C), jnp.int32),
        ] + [pltpu.VMEM((_C, _D), jnp.float32)] * _NB + [
            pltpu.VMEM_SHARED((_NP, _D), jnp.float32),
        ] + [pltpu.SemaphoreType.DMA] * (2 * _NB),
    )(f, src2, dst2, zrows)


def _row_spec(width=_D):
    return pl.BlockSpec((_BR, width), lambda i: (i, 0))


def _full_spec(shape):
    nd = len(shape)
    return pl.BlockSpec(shape, lambda i: (0,) * nd)


def _k2(x_pad, degpart, w, b):
    def body(x_ref, dp_ref, w_ref, b_ref, u_ref, dinv_ref):
        i = pl.program_id(0)
        deg = dp_ref[0, :, 0:1] + dp_ref[1, :, 0:1] + 1.0
        rows = lax.broadcasted_iota(jnp.int32, (_BR, 1), 0) + i * _BR
        dinv = jnp.where(rows < _N, lax.rsqrt(deg), 0.0)
        phi = jnp.maximum(
            jnp.dot(x_ref[...], w_ref[...], preferred_element_type=jnp.float32)
            + b_ref[...][None, :], 0.0)
        u_ref[...] = dinv * phi
        dinv_ref[...] = dinv

    return pl.pallas_call(
        body,
        grid=(_NP // _BR,),
        in_specs=[
            _row_spec(_DIN),
            pl.BlockSpec((_NC, _BR, _D), lambda i: (0, i, 0)),
            _full_spec((_DIN, _D)),
            _full_spec((_D,)),
        ],
        out_specs=[_row_spec(_D), _row_spec(1)],
        out_shape=[
            jax.ShapeDtypeStruct((_NP, _D), jnp.float32),
            jax.ShapeDtypeStruct((_NP, 1), jnp.float32),
        ],
    )(x_pad, degpart, w, b)


def _k4(acc, u1, dinv, w, b):
    def body(a_ref, u_ref, d_ref, w_ref, b_ref, q_ref, e_ref):
        d = d_ref[...]
        q = d * (a_ref[0] + a_ref[1] + u_ref[...])
        enc = jnp.dot(q, w_ref[...], preferred_element_type=jnp.float32) + b_ref[...][None, :]
        q_ref[...] = q
        e_ref[...] = d * enc

    return pl.pallas_call(
        body,
        grid=(_NP // _BR,),
        in_specs=[
            pl.BlockSpec((_NC, _BR, _D), lambda i: (0, i, 0)),
            _row_spec(_D),
            _row_spec(1),
            _full_spec((_D, _D)),
            _full_spec((_D,)),
        ],
        out_specs=[_row_spec(_D), _row_spec(_D)],
        out_shape=[
            jax.ShapeDtypeStruct((_NP, _D), jnp.float32),
            jax.ShapeDtypeStruct((_NP, _D), jnp.float32),
        ],
    )(acc, u1, dinv, w, b)


def _k6(acc, e1, dinv, eps, wm, bm, ws, bs, wz, bz):
    def body(a_ref, e_ref, d_ref, eps_ref, wm_ref, bm_ref, ws_ref, bs_ref,
             wz_ref, bz_ref, mean_ref, std_ref, z_ref, p_ref):
        d = d_ref[...]
        r = d * (a_ref[0] + a_ref[1] + e_ref[...])
        m = jnp.dot(r, wm_ref[...], preferred_element_type=jnp.float32) + bm_ref[...][None, :]
        s = jax.nn.softplus(
            jnp.dot(r, ws_ref[...], preferred_element_type=jnp.float32) + bs_ref[...][None, :])
        z = eps_ref[...] * s + m
        pz = jnp.maximum(
            jnp.dot(z, wz_ref[...], preferred_element_type=jnp.float32) + bz_ref[...][None, :],
            0.0)
        mean_ref[...] = m
        std_ref[...] = s
        z_ref[...] = z
        p_ref[...] = d * pz

    return pl.pallas_call(
        body,
        grid=(_NP // _BR,),
        in_specs=[
            pl.BlockSpec((_NC, _BR, _D), lambda i: (0, i, 0)),
            _row_spec(_D),
            _row_spec(1),
            _row_spec(_D),
            _full_spec((_D, _D)),
            _full_spec((_D,)),
            _full_spec((_D, _D)),
            _full_spec((_D,)),
            _full_spec((_D, _D)),
            _full_spec((_D,)),
        ],
        out_specs=[_row_spec(_D)] * 4,
        out_shape=[jax.ShapeDtypeStruct((_NP, _D), jnp.float32)] * 4,
    )(acc, e1, dinv, eps, wm, bm, ws, bs, wz, bz)


def _k8(acc, p1, dinv, q, wxz_t, wxz_b, wxh_t, wxh_b, bxz, bhz, bxh, bhh,
        prior_b, wpm, bpm, wps, bps):
    def body(a_ref, p_ref, d_ref, q_ref, wzt_ref, wzb_ref, wht_ref, whb_ref,
             bxz_ref, bhz_ref, bxh_ref, bhh_ref, pb_ref, wpm_ref, bpm_ref,
             wps_ref, bps_ref, h_ref, pm_ref, ps_ref):
        d = d_ref[...]
        t = d * (a_ref[0] + a_ref[1] + p_ref[...])
        qv = q_ref[...]
        ga = (jnp.dot(qv, wzt_ref[...], preferred_element_type=jnp.float32)
              + jnp.dot(t, wzb_ref[...], preferred_element_type=jnp.float32)
              + (bxz_ref[...] + bhz_ref[...])[None, :])
        z_g = jax.nn.sigmoid(ga)
        ha = (jnp.dot(qv, wht_ref[...], preferred_element_type=jnp.float32)
              + jnp.dot(t, whb_ref[...], preferred_element_type=jnp.float32)
              + (bxh_ref[...] + bhh_ref[...])[None, :])
        h_tilde = jnp.tanh(ha)
        h_ref[...] = (1.0 - z_g) * h_tilde
        pr = jnp.maximum(pb_ref[...], 0.0)[None, :]
        pm_row = jnp.dot(pr, wpm_ref[...], preferred_element_type=jnp.float32) + bpm_ref[...][None, :]
        ps_row = jax.nn.softplus(
            jnp.dot(pr, wps_ref[...], preferred_element_type=jnp.float32) + bps_ref[...][None, :])
        pm_ref[...] = jnp.broadcast_to(pm_row, (_BR, _D))
        ps_ref[...] = jnp.broadcast_to(ps_row, (_BR, _D))

    return pl.pallas_call(
        body,
        grid=(_NP // _BR,),
        in_specs=[
            pl.BlockSpec((_NC, _BR, _D), lambda i: (0, i, 0)),
            _row_spec(_D),
            _row_spec(1),
            _row_spec(_D),
            _full_spec((_D, _D)),
            _full_spec((_D, _D)),
            _full_spec((_D, _D)),
            _full_spec((_D, _D)),
            _full_spec((_D,)),
            _full_spec((_D,)),
            _full_spec((_D,)),
            _full_spec((_D,)),
            _full_spec((_D,)),
            _full_spec((_D, _D)),
            _full_spec((_D,)),
            _full_spec((_D, _D)),
            _full_spec((_D,)),
        ],
        out_specs=[_row_spec(_D)] * 3,
        out_shape=[jax.ShapeDtypeStruct((_NP, _D), jnp.float32)] * 3,
    )(acc, p1, dinv, q, wxz_t, wxz_b, wxh_t, wxh_b, bxz, bhz, bxh, bhh,
      prior_b, wpm, bpm, wps, bps)


_BRD = 400


def _k9(zs):
    def body(a_ref, b_ref, o_ref):
        o_ref[...] = jax.nn.sigmoid(
            lax.dot_general(a_ref[...], b_ref[...], (((1,), (1,)), ((), ())),
                            preferred_element_type=jnp.float32))

    return pl.pallas_call(
        body,
        grid=(_N // _BRD,),
        in_specs=[
            pl.BlockSpec((_BRD, _D), lambda i: (i, 0)),
            pl.BlockSpec((_N, _D), lambda i: (0, 0)),
        ],
        out_specs=pl.BlockSpec((_BRD, _N), lambda i: (i, 0)),
        out_shape=jax.ShapeDtypeStruct((_N, _N), jnp.float32),
    )(zs, zs)


def kernel(x, edge_index, phi_x_W, phi_x_b, phi_z_W, phi_z_b, enc_W, enc_b,
           enc_mean_W, enc_mean_b, enc_std_W, enc_std_b, prior_W, prior_b,
           prior_mean_W, prior_mean_b, prior_std_W, prior_std_b, rnn_xz_W,
           rnn_xz_b, rnn_hz_W, rnn_hz_b, rnn_xr_W, rnn_xr_b, rnn_hr_W,
           rnn_hr_b, rnn_xh_W, rnn_xh_b, rnn_hh_W, rnn_hh_b):
    f32 = jnp.float32
    src = edge_index[0].astype(jnp.int32)
    dst = edge_index[1].astype(jnp.int32)
    padv = jnp.full((_EP - _E,), _N, jnp.int32)
    src2 = jnp.concatenate([src, padv]).reshape(_TCH, _C)
    dst2 = jnp.concatenate([dst, padv]).reshape(_TCH, _C)
    zrows = jnp.zeros((_RT, _D), f32)
    zpad = jnp.zeros((_NP - _N, _D), f32)

    ones_rows = jnp.ones((_C, _D), f32)
    degpart = _sc_degree(dst2, ones_rows, zrows)

    x_pad = jnp.concatenate([x.astype(f32), zpad], axis=0)
    u1, dinv = _k2(x_pad, degpart, phi_x_W, phi_x_b)

    acc_a = _sc_scatter(u1, src2, dst2, zrows)
    q, e1 = _k4(acc_a, u1, dinv, enc_W[:_D], enc_b)

    acc_b = _sc_scatter(e1, src2, dst2, zrows)
    eps = jax.random.normal(jax.random.key(42), (_N, _D), dtype=f32)
    eps_pad = jnp.concatenate([eps, zpad], axis=0)
    enc_mean_p, enc_std_p, z_p, p1 = _k6(
        acc_b, e1, dinv, eps_pad, enc_mean_W, enc_mean_b, enc_std_W,
        enc_std_b, phi_z_W, phi_z_b)

    acc_c = _sc_scatter(p1, src2, dst2, zrows)
    h_p, pm_p, ps_p = _k8(
        acc_c, p1, dinv, q, rnn_xz_W[:_D], rnn_xz_W[_D:], rnn_xh_W[:_D],
        rnn_xh_W[_D:], rnn_xz_b, rnn_hz_b, rnn_xh_b, rnn_hh_b, prior_b,
        prior_mean_W, prior_mean_b, prior_std_W, prior_std_b)

    zs = z_p[:_N]
    dec = _k9(zs)

    return (dec, enc_mean_p[:_N], enc_std_p[:_N], pm_p[:_N], ps_p[:_N],
            h_p[:_N][None])


# speedup vs baseline: 9.7736x; 1.0114x over previous
import functools

import jax
import jax.numpy as jnp
from jax import lax
from jax.experimental import pallas as pl
from jax.experimental.pallas import tpu as pltpu
from jax.experimental.pallas import tpu_sc as plsc

_N = 10000
_DIN = 128
_D = 128
_E = 320000

_NC = 2
_NT = 16
_NW = _NC * _NT
_C = 128
_S = 80
_EP = _NW * _S * _C
_NP = 10112
_RT = _NP // _NT

_BR = 2528

_NB = 2
_TCH = _EP // _C

_S_FAST = 152
_S_SLOW = 8
_SUB_FAST = 48
_SUB_SLOW = 8
_FAST_CID = 0


def _sc_mesh():
    return plsc.VectorSubcoreMesh(core_axis_name="c", subcore_axis_name="s")


def _sc_degree(dst2, ones_rows, zrows):
    def body(dst_hbm, ones_hbm, z_hbm, out_hbm, dstv, onesv, acc):
        cid = lax.axis_index("c")
        sid = lax.axis_index("s")
        wid = sid * _NC + cid
        pltpu.sync_copy(z_hbm, acc.at[pl.ds(sid * _RT, _RT)])
        pltpu.sync_copy(ones_hbm, onesv)
        pltpu.sync_copy(dst_hbm.at[pl.ds(wid * _S, _S)], dstv)
        plsc.subcore_barrier()

        def step(k, carry):
            pltpu.sync_copy(onesv, acc.at[dstv.at[k]], add=True)
            return carry

        lax.fori_loop(0, _S, step, 0)
        plsc.subcore_barrier()
        pltpu.sync_copy(acc.at[pl.ds(sid * _RT, _RT)],
                        out_hbm.at[cid, pl.ds(sid * _RT, _RT)])

    return pl.kernel(
        body,
        out_type=jax.ShapeDtypeStruct((_NC, _NP, _D), jnp.float32),
        mesh=_sc_mesh(),
        scratch_types=[
            pltpu.VMEM((_S, _C), jnp.int32),
            pltpu.VMEM((_C, _D), jnp.float32),
            pltpu.VMEM_SHARED((_NP, _D), jnp.float32),
        ],
    )(dst2, ones_rows, zrows)


def _sc_scatter(f, src2, dst2, zrows):
    def body(f_hbm, src_hbm, dst_hbm, z_hbm, out_hbm, srcv, dstv,
             r0, r1, acc, g0, g1, s0, s1):
        rows = (r0, r1)
        gsem = (g0, g1)
        ssem = (s0, s1)
        cid = lax.axis_index("c")
        sid = lax.axis_index("s")
        fast = cid == _FAST_CID
        pltpu.sync_copy(z_hbm, acc.at[pl.ds(sid * _RT, _RT)])
        plsc.subcore_barrier()

        def run(n_chunks, sub, start):
            gh = sub // _NB

            def subslab(h, carry):
                off = pl.multiple_of(start + h * sub, 8)
                pltpu.sync_copy(src_hbm.at[pl.ds(off, sub)],
                                srcv.at[pl.ds(0, sub)])
                pltpu.sync_copy(dst_hbm.at[pl.ds(off, sub)],
                                dstv.at[pl.ds(0, sub)])
                for b in range(_NB):
                    pltpu.async_copy(f_hbm.at[srcv.at[b]], rows[b], gsem[b])

                def outer(g, c):
                    base = g * _NB
                    for b in range(_NB):
                        pltpu.make_async_copy(
                            f_hbm.at[srcv.at[base + b]], rows[b], gsem[b]).wait()
                        pltpu.async_copy(
                            rows[b], acc.at[dstv.at[base + b]], ssem[b], add=True)
                    for b in range(_NB):
                        pltpu.make_async_copy(
                            rows[b], acc.at[dstv.at[base + b]], ssem[b]).wait()

                        @pl.when(g < gh - 1)
                        def _():
                            pltpu.async_copy(
                                f_hbm.at[srcv.at[base + _NB + b]], rows[b],
                                gsem[b])

                    return c

                lax.fori_loop(0, gh, outer, 0)
                return carry

            lax.fori_loop(0, n_chunks // sub, subslab, 0)

        @pl.when(fast)
        def _():
            run(144, _SUB_FAST, sid * _S_FAST)
            run(_S_FAST - 144, _S_FAST - 144, sid * _S_FAST + 144)

        @pl.when(jnp.logical_not(fast))
        def _():
            run(_S_SLOW, _SUB_SLOW, _NT * _S_FAST + sid * _S_SLOW)

        plsc.subcore_barrier()
        pltpu.sync_copy(acc.at[pl.ds(sid * _RT, _RT)],
                        out_hbm.at[cid, pl.ds(sid * _RT, _RT)])

    return pl.kernel(
        body,
        out_type=jax.ShapeDtypeStruct((_NC, _NP, _D), jnp.float32),
        mesh=_sc_mesh(),
        scratch_types=[
            pltpu.VMEM((_SUB_FAST, _C), jnp.int32),
            pltpu.VMEM((_SUB_FAST, _C), jnp.int32),
        ] + [pltpu.VMEM((_C, _D), jnp.float32)] * _NB + [
            pltpu.VMEM_SHARED((_NP, _D), jnp.float32),
        ] + [pltpu.SemaphoreType.DMA] * (2 * _NB),
    )(f, src2, dst2, zrows)


def _row_spec(width=_D):
    return pl.BlockSpec((_BR, width), lambda i: (i, 0))


def _full_spec(shape):
    nd = len(shape)
    return pl.BlockSpec(shape, lambda i: (0,) * nd)


def _k2(x_pad, degpart, w, b):
    def body(x_ref, dp_ref, w_ref, b_ref, u_ref, dinv_ref):
        i = pl.program_id(0)
        deg = dp_ref[0, :, 0:1] + dp_ref[1, :, 0:1] + 1.0
        rows = lax.broadcasted_iota(jnp.int32, (_BR, 1), 0) + i * _BR
        dinv = jnp.where(rows < _N, lax.rsqrt(deg), 0.0)
        phi = jnp.maximum(
            jnp.dot(x_ref[...], w_ref[...], preferred_element_type=jnp.float32)
            + b_ref[...][None, :], 0.0)
        u_ref[...] = dinv * phi
        dinv_ref[...] = dinv

    return pl.pallas_call(
        body,
        grid=(_NP // _BR,),
        in_specs=[
            _row_spec(_DIN),
            pl.BlockSpec((_NC, _BR, _D), lambda i: (0, i, 0)),
            _full_spec((_DIN, _D)),
            _full_spec((_D,)),
        ],
        out_specs=[_row_spec(_D), _row_spec(1)],
        out_shape=[
            jax.ShapeDtypeStruct((_NP, _D), jnp.float32),
            jax.ShapeDtypeStruct((_NP, 1), jnp.float32),
        ],
    )(x_pad, degpart, w, b)


def _k4(acc, u1, dinv, w, b):
    def body(a_ref, u_ref, d_ref, w_ref, b_ref, q_ref, e_ref):
        d = d_ref[...]
        q = d * (a_ref[0] + a_ref[1] + u_ref[...])
        enc = jnp.dot(q, w_ref[...], preferred_element_type=jnp.float32) + b_ref[...][None, :]
        q_ref[...] = q
        e_ref[...] = d * enc

    return pl.pallas_call(
        body,
        grid=(_NP // _BR,),
        in_specs=[
            pl.BlockSpec((_NC, _BR, _D), lambda i: (0, i, 0)),
            _row_spec(_D),
            _row_spec(1),
            _full_spec((_D, _D)),
            _full_spec((_D,)),
        ],
        out_specs=[_row_spec(_D), _row_spec(_D)],
        out_shape=[
            jax.ShapeDtypeStruct((_NP, _D), jnp.float32),
            jax.ShapeDtypeStruct((_NP, _D), jnp.float32),
        ],
    )(acc, u1, dinv, w, b)


def _k6(acc, e1, dinv, eps, wm, bm, ws, bs, wz, bz):
    def body(a_ref, e_ref, d_ref, eps_ref, wm_ref, bm_ref, ws_ref, bs_ref,
             wz_ref, bz_ref, mean_ref, std_ref, z_ref, p_ref):
        d = d_ref[...]
        r = d * (a_ref[0] + a_ref[1] + e_ref[...])
        m = jnp.dot(r, wm_ref[...], preferred_element_type=jnp.float32) + bm_ref[...][None, :]
        s = jax.nn.softplus(
            jnp.dot(r, ws_ref[...], preferred_element_type=jnp.float32) + bs_ref[...][None, :])
        z = eps_ref[...] * s + m
        pz = jnp.maximum(
            jnp.dot(z, wz_ref[...], preferred_element_type=jnp.float32) + bz_ref[...][None, :],
            0.0)
        mean_ref[...] = m
        std_ref[...] = s
        z_ref[...] = z
        p_ref[...] = d * pz

    return pl.pallas_call(
        body,
        grid=(_NP // _BR,),
        in_specs=[
            pl.BlockSpec((_NC, _BR, _D), lambda i: (0, i, 0)),
            _row_spec(_D),
            _row_spec(1),
            _row_spec(_D),
            _full_spec((_D, _D)),
            _full_spec((_D,)),
            _full_spec((_D, _D)),
            _full_spec((_D,)),
            _full_spec((_D, _D)),
            _full_spec((_D,)),
        ],
        out_specs=[_row_spec(_D)] * 4,
        out_shape=[jax.ShapeDtypeStruct((_NP, _D), jnp.float32)] * 4,
    )(acc, e1, dinv, eps, wm, bm, ws, bs, wz, bz)


def _k8(acc, p1, dinv, q, wxz_t, wxz_b, wxh_t, wxh_b, bxz, bhz, bxh, bhh,
        prior_b, wpm, bpm, wps, bps):
    def body(a_ref, p_ref, d_ref, q_ref, wzt_ref, wzb_ref, wht_ref, whb_ref,
             bxz_ref, bhz_ref, bxh_ref, bhh_ref, pb_ref, wpm_ref, bpm_ref,
             wps_ref, bps_ref, h_ref, pm_ref, ps_ref):
        d = d_ref[...]
        t = d * (a_ref[0] + a_ref[1] + p_ref[...])
        qv = q_ref[...]
        ga = (jnp.dot(qv, wzt_ref[...], preferred_element_type=jnp.float32)
              + jnp.dot(t, wzb_ref[...], preferred_element_type=jnp.float32)
              + (bxz_ref[...] + bhz_ref[...])[None, :])
        z_g = jax.nn.sigmoid(ga)
        ha = (jnp.dot(qv, wht_ref[...], preferred_element_type=jnp.float32)
              + jnp.dot(t, whb_ref[...], preferred_element_type=jnp.float32)
              + (bxh_ref[...] + bhh_ref[...])[None, :])
        h_tilde = jnp.tanh(ha)
        h_ref[...] = (1.0 - z_g) * h_tilde
        pr = jnp.maximum(pb_ref[...], 0.0)[None, :]
        pm_row = jnp.dot(pr, wpm_ref[...], preferred_element_type=jnp.float32) + bpm_ref[...][None, :]
        ps_row = jax.nn.softplus(
            jnp.dot(pr, wps_ref[...], preferred_element_type=jnp.float32) + bps_ref[...][None, :])
        pm_ref[...] = jnp.broadcast_to(pm_row, (_BR, _D))
        ps_ref[...] = jnp.broadcast_to(ps_row, (_BR, _D))

    return pl.pallas_call(
        body,
        grid=(_NP // _BR,),
        in_specs=[
            pl.BlockSpec((_NC, _BR, _D), lambda i: (0, i, 0)),
            _row_spec(_D),
            _row_spec(1),
            _row_spec(_D),
            _full_spec((_D, _D)),
            _full_spec((_D, _D)),
            _full_spec((_D, _D)),
            _full_spec((_D, _D)),
            _full_spec((_D,)),
            _full_spec((_D,)),
            _full_spec((_D,)),
            _full_spec((_D,)),
            _full_spec((_D,)),
            _full_spec((_D, _D)),
            _full_spec((_D,)),
            _full_spec((_D, _D)),
            _full_spec((_D,)),
        ],
        out_specs=[_row_spec(_D)] * 3,
        out_shape=[jax.ShapeDtypeStruct((_NP, _D), jnp.float32)] * 3,
    )(acc, p1, dinv, q, wxz_t, wxz_b, wxh_t, wxh_b, bxz, bhz, bxh, bhh,
      prior_b, wpm, bpm, wps, bps)


_BRD = 400


def _k9(zs):
    def body(a_ref, b_ref, o_ref):
        o_ref[...] = jax.nn.sigmoid(
            lax.dot_general(a_ref[...], b_ref[...], (((1,), (1,)), ((), ())),
                            preferred_element_type=jnp.float32))

    return pl.pallas_call(
        body,
        grid=(_N // _BRD,),
        in_specs=[
            pl.BlockSpec((_BRD, _D), lambda i: (i, 0)),
            pl.BlockSpec((_N, _D), lambda i: (0, 0)),
        ],
        out_specs=pl.BlockSpec((_BRD, _N), lambda i: (i, 0)),
        out_shape=jax.ShapeDtypeStruct((_N, _N), jnp.float32),
    )(zs, zs)


def kernel(x, edge_index, phi_x_W, phi_x_b, phi_z_W, phi_z_b, enc_W, enc_b,
           enc_mean_W, enc_mean_b, enc_std_W, enc_std_b, prior_W, prior_b,
           prior_mean_W, prior_mean_b, prior_std_W, prior_std_b, rnn_xz_W,
           rnn_xz_b, rnn_hz_W, rnn_hz_b, rnn_xr_W, rnn_xr_b, rnn_hr_W,
           rnn_hr_b, rnn_xh_W, rnn_xh_b, rnn_hh_W, rnn_hh_b):
    f32 = jnp.float32
    src = edge_index[0].astype(jnp.int32)
    dst = edge_index[1].astype(jnp.int32)
    padv = jnp.full((_EP - _E,), _N, jnp.int32)
    src2 = jnp.concatenate([src, padv]).reshape(_TCH, _C)
    dst2 = jnp.concatenate([dst, padv]).reshape(_TCH, _C)
    zrows = jnp.zeros((_RT, _D), f32)
    zpad = jnp.zeros((_NP - _N, _D), f32)

    ones_rows = jnp.ones((_C, _D), f32)
    degpart = _sc_degree(dst2, ones_rows, zrows)

    x_pad = jnp.concatenate([x.astype(f32), zpad], axis=0)
    u1, dinv = _k2(x_pad, degpart, phi_x_W, phi_x_b)

    acc_a = _sc_scatter(u1, src2, dst2, zrows)
    q, e1 = _k4(acc_a, u1, dinv, enc_W[:_D], enc_b)

    acc_b = _sc_scatter(e1, src2, dst2, zrows)
    eps = jax.random.normal(jax.random.key(42), (_N, _D), dtype=f32)
    eps_pad = jnp.concatenate([eps, zpad], axis=0)
    enc_mean_p, enc_std_p, z_p, p1 = _k6(
        acc_b, e1, dinv, eps_pad, enc_mean_W, enc_mean_b, enc_std_W,
        enc_std_b, phi_z_W, phi_z_b)

    acc_c = _sc_scatter(p1, src2, dst2, zrows)
    h_p, pm_p, ps_p = _k8(
        acc_c, p1, dinv, q, rnn_xz_W[:_D], rnn_xz_W[_D:], rnn_xh_W[:_D],
        rnn_xh_W[_D:], rnn_xz_b, rnn_hz_b, rnn_xh_b, rnn_hh_b, prior_b,
        prior_mean_W, prior_mean_b, prior_std_W, prior_std_b)

    zs = z_p[:_N]
    dec = _k9(zs)

    return (dec, enc_mean_p[:_N], enc_std_p[:_N], pm_p[:_N], ps_p[:_N],
            h_p[:_N][None])
